# baseline (speedup 1.0000x reference)
import functools

import jax
import jax.numpy as jnp
from jax.experimental import pallas as pl
from jax.experimental.pallas import tpu as pltpu

H = 16
C = 32
TOPK = 8
G = 4


def _qkv_kernel(q_ref, k_ref, v_ref, wq_ref, wk_ref, wv_ref,
                bq_ref, bk_ref, bv_ref, o_ref):
    o_ref[0] = jnp.dot(q_ref[...].astype(jnp.bfloat16),
                       wq_ref[...].astype(jnp.bfloat16),
                       preferred_element_type=jnp.float32) + bq_ref[...]
    o_ref[1] = jnp.dot(k_ref[...].astype(jnp.bfloat16),
                       wk_ref[...].astype(jnp.bfloat16),
                       preferred_element_type=jnp.float32) + bk_ref[...]
    o_ref[2] = jnp.dot(v_ref[...].astype(jnp.bfloat16),
                       wv_ref[...].astype(jnp.bfloat16),
                       preferred_element_type=jnp.float32) + bv_ref[...]


def _qkv_proj(x_q, x_k, x_v, wqT, wkT, wvT, bq, bk, bv, bm,
              interpret=False):
    M, D = x_q.shape
    row = pl.BlockSpec((bm, D), lambda j: (j, 0))
    full = pl.BlockSpec((D, D), lambda j: (0, 0))
    vec = pl.BlockSpec((1, D), lambda j: (0, 0))
    return pl.pallas_call(
        _qkv_kernel,
        grid=(M // bm,),
        in_specs=[row, row, row, full, full, full, vec, vec, vec],
        out_specs=pl.BlockSpec((3, bm, D), lambda j: (0, j, 0)),
        out_shape=jax.ShapeDtypeStruct((3, M, D), jnp.float32),
        compiler_params=pltpu.CompilerParams(
            dimension_semantics=("parallel",)),
        interpret=interpret,
    )(x_q, x_k, x_v, wqT, wkT, wvT,
      bq.reshape(1, D), bk.reshape(1, D), bv.reshape(1, D))


def _first_max_mask_ax1(x):
    m = jnp.max(x, axis=1, keepdims=True)
    eq = x == m
    ii = jax.lax.broadcasted_iota(jnp.int32, x.shape, 1)
    big = jnp.where(eq, ii, x.shape[1])
    amin = jnp.min(big, axis=1, keepdims=True)
    return jnp.logical_and(eq, ii == amin), m


def _topk_softmax3(sc3):
    work = sc3
    selm = jnp.zeros(sc3.shape, dtype=jnp.bool_)
    gmax = None
    for _ in range(TOPK):
        f, m = _first_max_mask_ax1(work)
        gmax = m if gmax is None else gmax
        selm = jnp.logical_or(selm, f)
        work = jnp.where(f, -jnp.inf, work)
    e = jnp.where(selm, jnp.exp(sc3 - gmax), 0.0)
    return e / jnp.sum(e, axis=1, keepdims=True)


def _block_diag(mats):
    z = jnp.zeros(mats[0].shape, dtype=mats[0].dtype)
    rows = []
    for i, a in enumerate(mats):
        rows.append(jnp.concatenate(
            [a if j == i else z for j in range(len(mats))], axis=1))
    return jnp.concatenate(rows, axis=0)


def _fused_kernel(scale, hd, q_ref, k_ref, v_ref, c_ref, wo_ref, bo_ref,
                  aw_ref, o_ref):
    h = pl.program_id(1)
    qb = q_ref[0]
    kb = k_ref[0]
    vb = v_ref[0]
    S = kb.shape[0]

    lane = jax.lax.broadcasted_iota(jnp.int32, kb.shape, 1)
    head_of_lane = lane // hd
    sq = kb * kb
    nrm = jnp.zeros(kb.shape, dtype=jnp.float32)
    for i in range(G):
        sel = head_of_lane == i
        ni = jnp.sum(jnp.where(sel, sq, 0.0), axis=1, keepdims=True)
        nrm = jnp.where(sel, ni, nrm)
    nrm = jnp.maximum(jnp.sqrt(nrm), 1e-12)
    kn = kb / nrm

    cens = [c_ref[i] for i in range(G)]
    cns = [c / jnp.maximum(
        jnp.sqrt(jnp.sum(c * c, axis=-1, keepdims=True)), 1e-12)
        for c in cens]
    cnD = _block_diag(cns)

    simsT = jax.lax.dot_general(
        cnD, kn, (((1,), (1,)), ((), ())),
        preferred_element_type=jnp.float32)
    sims3 = simsT.reshape(G, C, S)

    oh3, _ = _first_max_mask_ax1(sims3)
    oh3 = oh3.astype(jnp.float32)
    counts = jnp.sum(oh3, axis=2, keepdims=True)
    ohT = oh3.reshape(G * C, S)
    ksumD = jax.lax.dot_general(
        ohT, kb, (((1,), (0,)), ((), ())),
        preferred_element_type=jnp.float32)
    vsumD = jax.lax.dot_general(
        ohT, vb, (((1,), (0,)), ((), ())),
        preferred_element_type=jnp.float32)

    has = counts > 0.0
    den = jnp.maximum(counts, 1.0)
    kmeans = []
    vmeans = []
    for i in range(G):
        ks = ksumD[C * i:C * (i + 1), hd * i:hd * (i + 1)]
        vs = vsumD[C * i:C * (i + 1), hd * i:hd * (i + 1)]
        kmeans.append(jnp.where(has[i], ks / den[i], cens[i]))
        vmeans.append(jnp.where(has[i], vs / den[i], 0.0))

    kmD = _block_diag(kmeans)
    scoresT = jax.lax.dot_general(
        kmD, qb, (((1,), (1,)), ((), ())),
        preferred_element_type=jnp.float32) * scale

    aw3 = _topk_softmax3(scoresT.reshape(G, C, S))

    awT = aw3.reshape(G * C, S)
    vmD = _block_diag(vmeans)
    y = jax.lax.dot_general(
        awT, vmD, (((0,), (0,)), ((), ())),
        preferred_element_type=jnp.float32)
    partial = jnp.dot(y.astype(jnp.bfloat16),
                      wo_ref[...].astype(jnp.bfloat16),
                      preferred_element_type=jnp.float32)

    for i in range(G):
        aw_ref[0, i] = aw3[i].T

    @pl.when(h == 0)
    def _():
        o_ref[0] = partial + bo_ref[...]

    @pl.when(h != 0)
    def _():
        o_ref[0] += partial


def _fused_attn(qkv, cen, woT, bo, B, S, scale, interpret=False):
    C_, hd = cen.shape[1], cen.shape[2]
    D = qkv.shape[2]
    return pl.pallas_call(
        functools.partial(_fused_kernel, scale, hd),
        grid=(B, H // G),
        in_specs=[
            pl.BlockSpec((1, S, G * hd), lambda b, h: (0, b, h)),
            pl.BlockSpec((1, S, G * hd), lambda b, h: (1, b, h)),
            pl.BlockSpec((1, S, G * hd), lambda b, h: (2, b, h)),
            pl.BlockSpec((G, C_, hd), lambda b, h: (h, 0, 0)),
            pl.BlockSpec((G * hd, D), lambda b, h: (h, 0)),
            pl.BlockSpec((1, D), lambda b, h: (0, 0)),
        ],
        out_specs=[
            pl.BlockSpec((1, G, S, C_), lambda b, h: (b, h, 0, 0)),
            pl.BlockSpec((1, S, D), lambda b, h: (b, 0, 0)),
        ],
        out_shape=[
            jax.ShapeDtypeStruct((B, H, S, C_), jnp.float32),
            jax.ShapeDtypeStruct((B, S, D), jnp.float32),
        ],
        compiler_params=pltpu.CompilerParams(
            dimension_semantics=("parallel", "arbitrary")),
        interpret=interpret,
    )(qkv, qkv, qkv, cen, woT, bo.reshape(1, D))


def _impl(query, key, value, Wq, bq, Wk, bk, Wv, bv, Wo, bo, centroids,
          interpret=False):
    B, Qlen, D = query.shape
    S = key.shape[1]
    hd = D // H
    scale = hd ** (-0.5)

    qkv = _qkv_proj(query.reshape(B * Qlen, D), key.reshape(B * S, D),
                    value.reshape(B * S, D), Wq.T, Wk.T, Wv.T, bq, bk, bv,
                    bm=512, interpret=interpret)
    attn, out = _fused_attn(qkv, centroids, Wo.T, bo, B, S, scale,
                            interpret=interpret)
    return out, attn


def kernel(query, key, value, Wq, bq, Wk, bk, Wv, bv, Wo, bo, centroids):
    return _impl(query, key, value, Wq, bq, Wk, bk, Wv, bv, Wo, bo,
                 centroids)


# speedup vs baseline: 1.0115x; 1.0115x over previous
import functools

import jax
import jax.numpy as jnp
from jax.experimental import pallas as pl
from jax.experimental.pallas import tpu as pltpu

H = 16
C = 32
TOPK = 8
G = 4


def _qkv_kernel(q_ref, k_ref, v_ref, wq_ref, wk_ref, wv_ref,
                bq_ref, bk_ref, bv_ref, o_ref, ov_ref):
    o_ref[0] = jnp.dot(q_ref[...], wq_ref[...],
                       preferred_element_type=jnp.float32) + bq_ref[...]
    o_ref[1] = jnp.dot(k_ref[...], wk_ref[...],
                       preferred_element_type=jnp.float32) + bk_ref[...]
    ov_ref[...] = (jnp.dot(v_ref[...], wv_ref[...],
                           preferred_element_type=jnp.float32)
                   + bv_ref[...]).astype(jnp.bfloat16)


def _qkv_proj(x_q, x_k, x_v, wqT, wkT, wvT, bq, bk, bv, bm,
              interpret=False):
    M, D = x_q.shape
    row = pl.BlockSpec((bm, D), lambda j: (j, 0))
    full = pl.BlockSpec((D, D), lambda j: (0, 0))
    vec = pl.BlockSpec((1, D), lambda j: (0, 0))
    return pl.pallas_call(
        _qkv_kernel,
        grid=(M // bm,),
        in_specs=[row, row, row, full, full, full, vec, vec, vec],
        out_specs=[
            pl.BlockSpec((2, bm, D), lambda j: (0, j, 0)),
            pl.BlockSpec((bm, D), lambda j: (j, 0)),
        ],
        out_shape=[
            jax.ShapeDtypeStruct((2, M, D), jnp.float32),
            jax.ShapeDtypeStruct((M, D), jnp.bfloat16),
        ],
        compiler_params=pltpu.CompilerParams(
            dimension_semantics=("parallel",)),
        interpret=interpret,
    )(x_q, x_k, x_v, wqT, wkT, wvT,
      bq.reshape(1, D), bk.reshape(1, D), bv.reshape(1, D))


def _first_max_mask_ax1(x):
    m = jnp.max(x, axis=1, keepdims=True)
    eq = x == m
    ii = jax.lax.broadcasted_iota(jnp.int32, x.shape, 1)
    big = jnp.where(eq, ii, x.shape[1])
    amin = jnp.min(big, axis=1, keepdims=True)
    return jnp.logical_and(eq, ii == amin), m


def _topk_softmax3(sc3):
    work = sc3
    selm = jnp.zeros(sc3.shape, dtype=jnp.bool_)
    gmax = None
    for _ in range(TOPK):
        f, m = _first_max_mask_ax1(work)
        gmax = m if gmax is None else gmax
        selm = jnp.logical_or(selm, f)
        work = jnp.where(f, -jnp.inf, work)
    e = jnp.where(selm, jnp.exp(sc3 - gmax), 0.0)
    return e / jnp.sum(e, axis=1, keepdims=True)


def _block_diag(mats):
    z = jnp.zeros(mats[0].shape, dtype=mats[0].dtype)
    rows = []
    for i, a in enumerate(mats):
        rows.append(jnp.concatenate(
            [a if j == i else z for j in range(len(mats))], axis=1))
    return jnp.concatenate(rows, axis=0)


def _fused_kernel(scale, hd, q_ref, k_ref, v_ref, c_ref, wo_ref, bo_ref,
                  aw_ref, o_ref):
    h = pl.program_id(1)
    qb = q_ref[0]
    kb = k_ref[0]
    vb = v_ref[...].astype(jnp.float32)
    S = kb.shape[0]

    lane = jax.lax.broadcasted_iota(jnp.int32, kb.shape, 1)
    head_of_lane = lane // hd
    sq = kb * kb
    nrm = jnp.zeros(kb.shape, dtype=jnp.float32)
    for i in range(G):
        sel = head_of_lane == i
        ni = jnp.sum(jnp.where(sel, sq, 0.0), axis=1, keepdims=True)
        nrm = jnp.where(sel, ni, nrm)
    nrm = jnp.maximum(jnp.sqrt(nrm), 1e-12)
    kn = kb / nrm

    cens = [c_ref[i] for i in range(G)]
    cns = [c / jnp.maximum(
        jnp.sqrt(jnp.sum(c * c, axis=-1, keepdims=True)), 1e-12)
        for c in cens]
    cnD = _block_diag(cns)

    simsT = jax.lax.dot_general(
        cnD, kn, (((1,), (1,)), ((), ())),
        preferred_element_type=jnp.float32)
    sims3 = simsT.reshape(G, C, S)

    oh3, _ = _first_max_mask_ax1(sims3)
    oh3 = oh3.astype(jnp.float32)
    counts = jnp.sum(oh3, axis=2, keepdims=True)
    ohT = oh3.reshape(G * C, S)
    ksumD = jax.lax.dot_general(
        ohT, kb, (((1,), (0,)), ((), ())),
        preferred_element_type=jnp.float32)
    vsumD = jax.lax.dot_general(
        ohT, vb, (((1,), (0,)), ((), ())),
        preferred_element_type=jnp.float32)

    has = counts > 0.0
    den = jnp.maximum(counts, 1.0)
    kmeans = []
    vmeans = []
    for i in range(G):
        ks = ksumD[C * i:C * (i + 1), hd * i:hd * (i + 1)]
        vs = vsumD[C * i:C * (i + 1), hd * i:hd * (i + 1)]
        kmeans.append(jnp.where(has[i], ks / den[i], cens[i]))
        vmeans.append(jnp.where(has[i], vs / den[i], 0.0))

    kmD = _block_diag(kmeans)
    scoresT = jax.lax.dot_general(
        kmD, qb, (((1,), (1,)), ((), ())),
        preferred_element_type=jnp.float32) * scale

    aw3 = _topk_softmax3(scoresT.reshape(G, C, S))

    awT = aw3.reshape(G * C, S)
    vmD = _block_diag(vmeans)
    y = jax.lax.dot_general(
        awT, vmD, (((0,), (0,)), ((), ())),
        preferred_element_type=jnp.float32)
    partial = jnp.dot(y.astype(jnp.bfloat16),
                      wo_ref[...].astype(jnp.bfloat16),
                      preferred_element_type=jnp.float32)

    for i in range(G):
        aw_ref[0, i] = aw3[i].T

    @pl.when(h == 0)
    def _():
        o_ref[0] = partial + bo_ref[...]

    @pl.when(h != 0)
    def _():
        o_ref[0] += partial


def _fused_attn(qk, vproj, cen, woT, bo, B, S, scale, interpret=False):
    C_, hd = cen.shape[1], cen.shape[2]
    D = qk.shape[2]
    return pl.pallas_call(
        functools.partial(_fused_kernel, scale, hd),
        grid=(B, H // G),
        in_specs=[
            pl.BlockSpec((1, S, G * hd), lambda b, h: (0, b, h)),
            pl.BlockSpec((1, S, G * hd), lambda b, h: (1, b, h)),
            pl.BlockSpec((S, G * hd), lambda b, h: (b, h)),
            pl.BlockSpec((G, C_, hd), lambda b, h: (h, 0, 0)),
            pl.BlockSpec((G * hd, D), lambda b, h: (h, 0)),
            pl.BlockSpec((1, D), lambda b, h: (0, 0)),
        ],
        out_specs=[
            pl.BlockSpec((1, G, S, C_), lambda b, h: (b, h, 0, 0)),
            pl.BlockSpec((1, S, D), lambda b, h: (b, 0, 0)),
        ],
        out_shape=[
            jax.ShapeDtypeStruct((B, H, S, C_), jnp.float32),
            jax.ShapeDtypeStruct((B, S, D), jnp.float32),
        ],
        compiler_params=pltpu.CompilerParams(
            dimension_semantics=("parallel", "arbitrary")),
        interpret=interpret,
    )(qk, qk, vproj, cen, woT, bo.reshape(1, D))


def _impl(query, key, value, Wq, bq, Wk, bk, Wv, bv, Wo, bo, centroids,
          interpret=False):
    B, Qlen, D = query.shape
    S = key.shape[1]
    hd = D // H
    scale = hd ** (-0.5)

    qk, vproj = _qkv_proj(query.reshape(B * Qlen, D),
                          key.reshape(B * S, D),
                          value.reshape(B * S, D), Wq.T, Wk.T, Wv.T,
                          bq, bk, bv, bm=512, interpret=interpret)
    attn, out = _fused_attn(qk, vproj, centroids, Wo.T, bo, B, S, scale,
                            interpret=interpret)
    return out, attn


def kernel(query, key, value, Wq, bq, Wk, bk, Wv, bv, Wo, bo, centroids):
    return _impl(query, key, value, Wq, bq, Wk, bk, Wv, bv, Wo, bo,
                 centroids)


# speedup vs baseline: 1.1020x; 1.0895x over previous
import functools

import jax
import jax.numpy as jnp
from jax.experimental import pallas as pl
from jax.experimental.pallas import tpu as pltpu

H = 16
C = 32
TOPK = 8
G = 4


def _qkv_kernel(q_ref, k_ref, v_ref, wq_ref, wk_ref, wv_ref,
                bq_ref, bk_ref, bv_ref, o_ref, ov_ref):
    o_ref[0] = jnp.dot(q_ref[...], wq_ref[...],
                       preferred_element_type=jnp.float32) + bq_ref[...]
    o_ref[1] = jnp.dot(k_ref[...], wk_ref[...],
                       preferred_element_type=jnp.float32) + bk_ref[...]
    ov_ref[...] = (jnp.dot(v_ref[...], wv_ref[...],
                           preferred_element_type=jnp.float32)
                   + bv_ref[...]).astype(jnp.bfloat16)


def _qkv_proj(x_q, x_k, x_v, wqT, wkT, wvT, bq, bk, bv, bm,
              interpret=False):
    M, D = x_q.shape
    row = pl.BlockSpec((bm, D), lambda j: (j, 0))
    full = pl.BlockSpec((D, D), lambda j: (0, 0))
    vec = pl.BlockSpec((1, D), lambda j: (0, 0))
    return pl.pallas_call(
        _qkv_kernel,
        grid=(M // bm,),
        in_specs=[row, row, row, full, full, full, vec, vec, vec],
        out_specs=[
            pl.BlockSpec((2, bm, D), lambda j: (0, j, 0)),
            pl.BlockSpec((bm, D), lambda j: (j, 0)),
        ],
        out_shape=[
            jax.ShapeDtypeStruct((2, M, D), jnp.float32),
            jax.ShapeDtypeStruct((M, D), jnp.bfloat16),
        ],
        compiler_params=pltpu.CompilerParams(
            dimension_semantics=("parallel",)),
        interpret=interpret,
    )(x_q, x_k, x_v, wqT, wkT, wvT,
      bq.reshape(1, D), bk.reshape(1, D), bv.reshape(1, D))


def _rank_key(x):
    b = jax.lax.bitcast_convert_type(x, jnp.int32)
    k = jnp.where(b >= 0, b, b ^ jnp.int32(0x7FFFFFFF))
    ii = jax.lax.broadcasted_iota(jnp.int32, x.shape, 1)
    return (k & jnp.int32(-32)) | (jnp.int32(C - 1) - ii)


def _first_max_mask_ax1(x):
    key = _rank_key(x)
    m = jnp.max(key, axis=1, keepdims=True)
    return key == m


def _topk_softmax3(sc3):
    gmax = jnp.max(sc3, axis=1, keepdims=True)
    work = _rank_key(sc3)
    neg = jnp.int32(-2147483648)
    selm = jnp.zeros(sc3.shape, dtype=jnp.bool_)
    for _ in range(TOPK):
        m = jnp.max(work, axis=1, keepdims=True)
        f = work == m
        selm = jnp.logical_or(selm, f)
        work = jnp.where(f, neg, work)
    e = jnp.where(selm, jnp.exp(sc3 - gmax), 0.0)
    return e / jnp.sum(e, axis=1, keepdims=True)


def _block_diag(mats):
    z = jnp.zeros(mats[0].shape, dtype=mats[0].dtype)
    rows = []
    for i, a in enumerate(mats):
        rows.append(jnp.concatenate(
            [a if j == i else z for j in range(len(mats))], axis=1))
    return jnp.concatenate(rows, axis=0)


def _fused_kernel(scale, hd, q_ref, k_ref, v_ref, c_ref, wo_ref, bo_ref,
                  aw_ref, o_ref):
    h = pl.program_id(1)
    qb = q_ref[0]
    kb = k_ref[0]
    vb = v_ref[...].astype(jnp.float32)
    S = kb.shape[0]

    lane = jax.lax.broadcasted_iota(jnp.int32, kb.shape, 1)
    head_of_lane = lane // hd
    sq = kb * kb
    nrm = jnp.zeros(kb.shape, dtype=jnp.float32)
    for i in range(G):
        sel = head_of_lane == i
        ni = jnp.sum(jnp.where(sel, sq, 0.0), axis=1, keepdims=True)
        nrm = jnp.where(sel, ni, nrm)
    nrm = jnp.maximum(jnp.sqrt(nrm), 1e-12)
    kn = kb / nrm

    cens = [c_ref[i] for i in range(G)]
    cns = [c / jnp.maximum(
        jnp.sqrt(jnp.sum(c * c, axis=-1, keepdims=True)), 1e-12)
        for c in cens]
    cnD = _block_diag(cns)

    simsT = jax.lax.dot_general(
        cnD, kn, (((1,), (1,)), ((), ())),
        preferred_element_type=jnp.float32)
    sims3 = simsT.reshape(G, C, S)

    oh3 = _first_max_mask_ax1(sims3).astype(jnp.float32)
    counts = jnp.sum(oh3, axis=2, keepdims=True)
    ohT = oh3.reshape(G * C, S)
    ksumD = jax.lax.dot_general(
        ohT, kb, (((1,), (0,)), ((), ())),
        preferred_element_type=jnp.float32)
    vsumD = jax.lax.dot_general(
        ohT, vb, (((1,), (0,)), ((), ())),
        preferred_element_type=jnp.float32)

    has = counts > 0.0
    den = jnp.maximum(counts, 1.0)
    kmeans = []
    vmeans = []
    for i in range(G):
        ks = ksumD[C * i:C * (i + 1), hd * i:hd * (i + 1)]
        vs = vsumD[C * i:C * (i + 1), hd * i:hd * (i + 1)]
        kmeans.append(jnp.where(has[i], ks / den[i], cens[i]))
        vmeans.append(jnp.where(has[i], vs / den[i], 0.0))

    kmD = _block_diag(kmeans)
    scoresT = jax.lax.dot_general(
        kmD, qb, (((1,), (1,)), ((), ())),
        preferred_element_type=jnp.float32) * scale

    aw3 = _topk_softmax3(scoresT.reshape(G, C, S))

    awT = aw3.reshape(G * C, S)
    vmD = _block_diag(vmeans)
    y = jax.lax.dot_general(
        awT, vmD, (((0,), (0,)), ((), ())),
        preferred_element_type=jnp.float32)
    partial = jnp.dot(y.astype(jnp.bfloat16),
                      wo_ref[...].astype(jnp.bfloat16),
                      preferred_element_type=jnp.float32)

    for i in range(G):
        aw_ref[0, i] = aw3[i].T

    @pl.when(h == 0)
    def _():
        o_ref[0] = partial + bo_ref[...]

    @pl.when(h != 0)
    def _():
        o_ref[0] += partial


def _fused_attn(qk, vproj, cen, woT, bo, B, S, scale, interpret=False):
    C_, hd = cen.shape[1], cen.shape[2]
    D = qk.shape[2]
    return pl.pallas_call(
        functools.partial(_fused_kernel, scale, hd),
        grid=(B, H // G),
        in_specs=[
            pl.BlockSpec((1, S, G * hd), lambda b, h: (0, b, h)),
            pl.BlockSpec((1, S, G * hd), lambda b, h: (1, b, h)),
            pl.BlockSpec((S, G * hd), lambda b, h: (b, h)),
            pl.BlockSpec((G, C_, hd), lambda b, h: (h, 0, 0)),
            pl.BlockSpec((G * hd, D), lambda b, h: (h, 0)),
            pl.BlockSpec((1, D), lambda b, h: (0, 0)),
        ],
        out_specs=[
            pl.BlockSpec((1, G, S, C_), lambda b, h: (b, h, 0, 0)),
            pl.BlockSpec((1, S, D), lambda b, h: (b, 0, 0)),
        ],
        out_shape=[
            jax.ShapeDtypeStruct((B, H, S, C_), jnp.float32),
            jax.ShapeDtypeStruct((B, S, D), jnp.float32),
        ],
        compiler_params=pltpu.CompilerParams(
            dimension_semantics=("parallel", "arbitrary")),
        interpret=interpret,
    )(qk, qk, vproj, cen, woT, bo.reshape(1, D))


def _impl(query, key, value, Wq, bq, Wk, bk, Wv, bv, Wo, bo, centroids,
          interpret=False):
    B, Qlen, D = query.shape
    S = key.shape[1]
    hd = D // H
    scale = hd ** (-0.5)

    qk, vproj = _qkv_proj(query.reshape(B * Qlen, D),
                          key.reshape(B * S, D),
                          value.reshape(B * S, D), Wq.T, Wk.T, Wv.T,
                          bq, bk, bv, bm=512, interpret=interpret)
    attn, out = _fused_attn(qk, vproj, centroids, Wo.T, bo, B, S, scale,
                            interpret=interpret)
    return out, attn


def kernel(query, key, value, Wq, bq, Wk, bk, Wv, bv, Wo, bo, centroids):
    return _impl(query, key, value, Wq, bq, Wk, bk, Wv, bv, Wo, bo,
                 centroids)


# speedup vs baseline: 1.1095x; 1.0068x over previous
import functools

import jax
import jax.numpy as jnp
from jax.experimental import pallas as pl
from jax.experimental.pallas import tpu as pltpu

H = 16
C = 32
TOPK = 8
G = 4


def _qkv_kernel(q_ref, k_ref, v_ref, wq_ref, wk_ref, wv_ref,
                bq_ref, bk_ref, bv_ref, o_ref, ov_ref):
    o_ref[0] = jnp.dot(q_ref[...], wq_ref[...],
                       preferred_element_type=jnp.float32) + bq_ref[...]
    o_ref[1] = jnp.dot(k_ref[...], wk_ref[...],
                       preferred_element_type=jnp.float32) + bk_ref[...]
    ov_ref[...] = (jnp.dot(v_ref[...], wv_ref[...],
                           preferred_element_type=jnp.float32)
                   + bv_ref[...]).astype(jnp.bfloat16)


def _qkv_proj(x_q, x_k, x_v, wqT, wkT, wvT, bq, bk, bv, bm,
              interpret=False):
    M, D = x_q.shape
    row = pl.BlockSpec((bm, D), lambda j: (j, 0))
    full = pl.BlockSpec((D, D), lambda j: (0, 0))
    vec = pl.BlockSpec((1, D), lambda j: (0, 0))
    return pl.pallas_call(
        _qkv_kernel,
        grid=(M // bm,),
        in_specs=[row, row, row, full, full, full, vec, vec, vec],
        out_specs=[
            pl.BlockSpec((2, bm, D), lambda j: (0, j, 0)),
            pl.BlockSpec((bm, D), lambda j: (j, 0)),
        ],
        out_shape=[
            jax.ShapeDtypeStruct((2, M, D), jnp.float32),
            jax.ShapeDtypeStruct((M, D), jnp.bfloat16),
        ],
        compiler_params=pltpu.CompilerParams(
            dimension_semantics=("parallel",)),
        interpret=interpret,
    )(x_q, x_k, x_v, wqT, wkT, wvT,
      bq.reshape(1, D), bk.reshape(1, D), bv.reshape(1, D))


def _rank_key(x):
    b = jax.lax.bitcast_convert_type(x, jnp.int32)
    k = jnp.where(b >= 0, b, b ^ jnp.int32(0x7FFFFFFF))
    ii = jax.lax.broadcasted_iota(jnp.int32, x.shape, 1)
    return (k & jnp.int32(-32)) | (jnp.int32(C - 1) - ii)


def _first_max_mask_ax1(x):
    key = _rank_key(x)
    m = jnp.max(key, axis=1, keepdims=True)
    return key == m


def _topk_softmax3(sc3):
    gmax = jnp.max(sc3, axis=1, keepdims=True)
    work = _rank_key(sc3)
    neg = jnp.int32(-2147483648)
    selm = jnp.zeros(sc3.shape, dtype=jnp.bool_)
    for _ in range(TOPK):
        m = jnp.max(work, axis=1, keepdims=True)
        f = work == m
        selm = jnp.logical_or(selm, f)
        work = jnp.where(f, neg, work)
    e = jnp.where(selm, jnp.exp(sc3 - gmax), 0.0)
    return e / jnp.sum(e, axis=1, keepdims=True)


def _block_diag(mats):
    z = jnp.zeros(mats[0].shape, dtype=mats[0].dtype)
    rows = []
    for i, a in enumerate(mats):
        rows.append(jnp.concatenate(
            [a if j == i else z for j in range(len(mats))], axis=1))
    return jnp.concatenate(rows, axis=0)


def _fused_kernel(scale, hd, q_ref, k_ref, v_ref, c_ref, wo_ref, bo_ref,
                  aw_ref, o_ref):
    h = pl.program_id(1)
    qb = q_ref[0]
    kb = k_ref[0]
    vb = v_ref[...]
    S = kb.shape[0]

    lane = jax.lax.broadcasted_iota(jnp.int32, kb.shape, 1)
    head_of_lane = lane // hd
    sq = kb * kb
    nrm = jnp.zeros(kb.shape, dtype=jnp.float32)
    for i in range(G):
        sel = head_of_lane == i
        ni = jnp.sum(jnp.where(sel, sq, 0.0), axis=1, keepdims=True)
        nrm = jnp.where(sel, ni, nrm)
    nrm = jnp.maximum(jnp.sqrt(nrm), 1e-12)
    kn = kb / nrm

    cens = [c_ref[i] for i in range(G)]
    cns = [c / jnp.maximum(
        jnp.sqrt(jnp.sum(c * c, axis=-1, keepdims=True)), 1e-12)
        for c in cens]
    cnD = _block_diag(cns)

    simsT = jax.lax.dot_general(
        cnD, kn, (((1,), (1,)), ((), ())),
        preferred_element_type=jnp.float32)
    sims3 = simsT.reshape(G, C, S)

    oh3 = _first_max_mask_ax1(sims3).astype(jnp.float32)
    counts = jnp.sum(oh3, axis=2, keepdims=True)
    ohT = oh3.reshape(G * C, S)
    ksumD = jax.lax.dot_general(
        ohT, kb, (((1,), (0,)), ((), ())),
        preferred_element_type=jnp.float32)
    vsumD = jax.lax.dot_general(
        ohT.astype(jnp.bfloat16), vb, (((1,), (0,)), ((), ())),
        preferred_element_type=jnp.float32)

    has = counts > 0.0
    den = jnp.maximum(counts, 1.0)
    kmeans = []
    vmeans = []
    for i in range(G):
        ks = ksumD[C * i:C * (i + 1), hd * i:hd * (i + 1)]
        vs = vsumD[C * i:C * (i + 1), hd * i:hd * (i + 1)]
        kmeans.append(jnp.where(has[i], ks / den[i], cens[i]))
        vmeans.append(jnp.where(has[i], vs / den[i], 0.0))

    kmD = _block_diag(kmeans)
    scoresT = jax.lax.dot_general(
        kmD, qb, (((1,), (1,)), ((), ())),
        preferred_element_type=jnp.float32) * scale

    aw3 = _topk_softmax3(scoresT.reshape(G, C, S))

    awT = aw3.reshape(G * C, S)
    vmD = _block_diag(vmeans)
    y = jax.lax.dot_general(
        awT, vmD, (((0,), (0,)), ((), ())),
        preferred_element_type=jnp.float32)
    partial = jnp.dot(y.astype(jnp.bfloat16), wo_ref[...],
                      preferred_element_type=jnp.float32)

    for i in range(G):
        aw_ref[0, i] = aw3[i].T

    @pl.when(h == 0)
    def _():
        o_ref[0] = partial + bo_ref[...]

    @pl.when(h != 0)
    def _():
        o_ref[0] += partial


def _fused_attn(qk, vproj, cen, woT, bo, B, S, scale, interpret=False):
    C_, hd = cen.shape[1], cen.shape[2]
    D = qk.shape[2]
    return pl.pallas_call(
        functools.partial(_fused_kernel, scale, hd),
        grid=(B, H // G),
        in_specs=[
            pl.BlockSpec((1, S, G * hd), lambda b, h: (0, b, h)),
            pl.BlockSpec((1, S, G * hd), lambda b, h: (1, b, h)),
            pl.BlockSpec((S, G * hd), lambda b, h: (b, h)),
            pl.BlockSpec((G, C_, hd), lambda b, h: (h, 0, 0)),
            pl.BlockSpec((G * hd, D), lambda b, h: (h, 0)),
            pl.BlockSpec((1, D), lambda b, h: (0, 0)),
        ],
        out_specs=[
            pl.BlockSpec((1, G, S, C_), lambda b, h: (b, h, 0, 0)),
            pl.BlockSpec((1, S, D), lambda b, h: (b, 0, 0)),
        ],
        out_shape=[
            jax.ShapeDtypeStruct((B, H, S, C_), jnp.float32),
            jax.ShapeDtypeStruct((B, S, D), jnp.float32),
        ],
        compiler_params=pltpu.CompilerParams(
            dimension_semantics=("parallel", "arbitrary")),
        interpret=interpret,
    )(qk, qk, vproj, cen, woT, bo.reshape(1, D))


def _impl(query, key, value, Wq, bq, Wk, bk, Wv, bv, Wo, bo, centroids,
          interpret=False):
    B, Qlen, D = query.shape
    S = key.shape[1]
    hd = D // H
    scale = hd ** (-0.5)

    qk, vproj = _qkv_proj(query.reshape(B * Qlen, D),
                          key.reshape(B * S, D),
                          value.reshape(B * S, D), Wq.T, Wk.T, Wv.T,
                          bq, bk, bv, bm=512, interpret=interpret)
    attn, out = _fused_attn(qk, vproj, centroids,
                            Wo.T.astype(jnp.bfloat16), bo, B, S, scale,
                            interpret=interpret)
    return out, attn


def kernel(query, key, value, Wq, bq, Wk, bk, Wv, bv, Wo, bo, centroids):
    return _impl(query, key, value, Wq, bq, Wk, bk, Wv, bv, Wo, bo,
                 centroids)


# speedup vs baseline: 1.1098x; 1.0003x over previous
import functools

import jax
import jax.numpy as jnp
from jax.experimental import pallas as pl
from jax.experimental.pallas import tpu as pltpu

H = 16
C = 32
TOPK = 8
G = 4


def _qkv_kernel(q_ref, k_ref, v_ref, wq_ref, wk_ref, wv_ref,
                bq_ref, bk_ref, bv_ref, o_ref, ov_ref):
    o_ref[0] = jnp.dot(q_ref[...], wq_ref[...],
                       preferred_element_type=jnp.float32) + bq_ref[...]
    o_ref[1] = jnp.dot(k_ref[...], wk_ref[...],
                       preferred_element_type=jnp.float32) + bk_ref[...]
    ov_ref[...] = (jnp.dot(v_ref[...], wv_ref[...],
                           preferred_element_type=jnp.float32)
                   + bv_ref[...]).astype(jnp.bfloat16)


def _qkv_proj(x_q, x_k, x_v, wqT, wkT, wvT, bq, bk, bv, bm,
              interpret=False):
    M, D = x_q.shape
    row = pl.BlockSpec((bm, D), lambda j: (j, 0))
    full = pl.BlockSpec((D, D), lambda j: (0, 0))
    vec = pl.BlockSpec((1, D), lambda j: (0, 0))
    return pl.pallas_call(
        _qkv_kernel,
        grid=(M // bm,),
        in_specs=[row, row, row, full, full, full, vec, vec, vec],
        out_specs=[
            pl.BlockSpec((2, bm, D), lambda j: (0, j, 0)),
            pl.BlockSpec((bm, D), lambda j: (j, 0)),
        ],
        out_shape=[
            jax.ShapeDtypeStruct((2, M, D), jnp.float32),
            jax.ShapeDtypeStruct((M, D), jnp.bfloat16),
        ],
        compiler_params=pltpu.CompilerParams(
            dimension_semantics=("parallel",)),
        interpret=interpret,
    )(x_q, x_k, x_v, wqT, wkT, wvT,
      bq.reshape(1, D), bk.reshape(1, D), bv.reshape(1, D))


def _rank_key(x):
    b = jax.lax.bitcast_convert_type(x, jnp.int32)
    k = jnp.where(b >= 0, b, b ^ jnp.int32(0x7FFFFFFF))
    ii = jax.lax.broadcasted_iota(jnp.int32, x.shape, 1)
    return (k & jnp.int32(-32)) | (jnp.int32(C - 1) - ii)


def _first_max_mask_ax1(x):
    key = _rank_key(x)
    m = jnp.max(key, axis=1, keepdims=True)
    return key == m


def _topk_softmax3(sc3):
    gmax = jnp.max(sc3, axis=1, keepdims=True)
    work = _rank_key(sc3)
    neg = jnp.int32(-2147483648)
    selm = jnp.zeros(sc3.shape, dtype=jnp.bool_)
    for _ in range(TOPK):
        m = jnp.max(work, axis=1, keepdims=True)
        f = work == m
        selm = jnp.logical_or(selm, f)
        work = jnp.where(f, neg, work)
    e = jnp.where(selm, jnp.exp(sc3 - gmax), 0.0)
    return e / jnp.sum(e, axis=1, keepdims=True)


def _block_diag(mats):
    z = jnp.zeros(mats[0].shape, dtype=mats[0].dtype)
    rows = []
    for i, a in enumerate(mats):
        rows.append(jnp.concatenate(
            [a if j == i else z for j in range(len(mats))], axis=1))
    return jnp.concatenate(rows, axis=0)


def _fused_kernel(scale, hd, q_ref, k_ref, v_ref, c_ref, wo_ref, bo_ref,
                  aw_ref, o_ref):
    h = pl.program_id(1)
    qb = q_ref[0]
    kb = k_ref[0]
    vb = v_ref[...]
    S = kb.shape[0]

    lane = jax.lax.broadcasted_iota(jnp.int32, kb.shape, 1)
    head_of_lane = lane // hd
    sq = kb * kb
    nrm = jnp.zeros(kb.shape, dtype=jnp.float32)
    for i in range(G):
        sel = head_of_lane == i
        ni = jnp.sum(jnp.where(sel, sq, 0.0), axis=1, keepdims=True)
        nrm = jnp.where(sel, ni, nrm)
    nrm = jnp.maximum(jnp.sqrt(nrm), 1e-12)
    kn = kb / nrm

    cens = [c_ref[i] for i in range(G)]
    cns = [c / jnp.maximum(
        jnp.sqrt(jnp.sum(c * c, axis=-1, keepdims=True)), 1e-12)
        for c in cens]
    cnD = _block_diag(cns)

    simsT = jax.lax.dot_general(
        cnD, kn, (((1,), (1,)), ((), ())),
        preferred_element_type=jnp.float32)
    sims3 = simsT.reshape(G, C, S)

    oh3 = _first_max_mask_ax1(sims3).astype(jnp.float32)
    counts = jnp.sum(oh3, axis=2, keepdims=True)
    ohT = oh3.reshape(G * C, S)
    ksumD = jax.lax.dot_general(
        ohT, kb, (((1,), (0,)), ((), ())),
        preferred_element_type=jnp.float32)
    vsumD = jax.lax.dot_general(
        ohT.astype(jnp.bfloat16), vb, (((1,), (0,)), ((), ())),
        preferred_element_type=jnp.float32)

    has = counts > 0.0
    den = jnp.maximum(counts, 1.0)
    kmeans = []
    vmeans = []
    for i in range(G):
        ks = ksumD[C * i:C * (i + 1), hd * i:hd * (i + 1)]
        vs = vsumD[C * i:C * (i + 1), hd * i:hd * (i + 1)]
        kmeans.append(jnp.where(has[i], ks / den[i], cens[i]))
        vmeans.append(jnp.where(has[i], vs / den[i], 0.0))

    kmD = _block_diag(kmeans)
    scoresT = jax.lax.dot_general(
        kmD, qb, (((1,), (1,)), ((), ())),
        preferred_element_type=jnp.float32) * scale

    aw3 = _topk_softmax3(scoresT.reshape(G, C, S))

    awT = aw3.reshape(G * C, S)
    vmD = _block_diag(vmeans)
    y = jax.lax.dot_general(
        awT, vmD, (((0,), (0,)), ((), ())),
        preferred_element_type=jnp.float32)
    partial = jnp.dot(y.astype(jnp.bfloat16), wo_ref[...],
                      preferred_element_type=jnp.float32)

    for i in range(G):
        aw_ref[0, i] = aw3[i].T

    @pl.when(h == 0)
    def _():
        o_ref[0] = partial + bo_ref[...]

    @pl.when(h != 0)
    def _():
        o_ref[0] += partial


def _fused_attn(qk, vproj, cen, woT, bo, B, S, scale, interpret=False):
    C_, hd = cen.shape[1], cen.shape[2]
    D = qk.shape[2]
    return pl.pallas_call(
        functools.partial(_fused_kernel, scale, hd),
        grid=(B, H // G),
        in_specs=[
            pl.BlockSpec((1, S, G * hd), lambda b, h: (0, b, h)),
            pl.BlockSpec((1, S, G * hd), lambda b, h: (1, b, h)),
            pl.BlockSpec((S, G * hd), lambda b, h: (b, h)),
            pl.BlockSpec((G, C_, hd), lambda b, h: (h, 0, 0)),
            pl.BlockSpec((G * hd, D), lambda b, h: (h, 0)),
            pl.BlockSpec((1, D), lambda b, h: (0, 0)),
        ],
        out_specs=[
            pl.BlockSpec((1, G, S, C_), lambda b, h: (b, h, 0, 0)),
            pl.BlockSpec((1, S, D), lambda b, h: (b, 0, 0)),
        ],
        out_shape=[
            jax.ShapeDtypeStruct((B, H, S, C_), jnp.float32),
            jax.ShapeDtypeStruct((B, S, D), jnp.float32),
        ],
        compiler_params=pltpu.CompilerParams(
            dimension_semantics=("parallel", "arbitrary")),
        interpret=interpret,
    )(qk, qk, vproj, cen, woT, bo.reshape(1, D))


def _impl(query, key, value, Wq, bq, Wk, bk, Wv, bv, Wo, bo, centroids,
          interpret=False):
    B, Qlen, D = query.shape
    S = key.shape[1]
    hd = D // H
    scale = hd ** (-0.5)

    qk, vproj = _qkv_proj(query.reshape(B * Qlen, D),
                          key.reshape(B * S, D),
                          value.reshape(B * S, D), Wq.T, Wk.T, Wv.T,
                          bq, bk, bv, bm=1024, interpret=interpret)
    attn, out = _fused_attn(qk, vproj, centroids,
                            Wo.T.astype(jnp.bfloat16), bo, B, S, scale,
                            interpret=interpret)
    return out, attn


def kernel(query, key, value, Wq, bq, Wk, bk, Wv, bv, Wo, bo, centroids):
    return _impl(query, key, value, Wq, bq, Wk, bk, Wv, bv, Wo, bo,
                 centroids)


# speedup vs baseline: 1.2595x; 1.1348x over previous
import functools

import jax
import jax.numpy as jnp
from jax.experimental import pallas as pl
from jax.experimental.pallas import tpu as pltpu

H = 16
C = 32
TOPK = 8
G = 4


def _mmT(x, w):
    return jax.lax.dot_general(
        x, w, (((1,), (1,)), ((), ())),
        preferred_element_type=jnp.float32)


def _qkv_kernel(q_ref, k_ref, v_ref, wq_ref, wk_ref, wv_ref,
                bq_ref, bk_ref, bv_ref, o_ref, ov_ref):
    o_ref[0] = _mmT(q_ref[...], wq_ref[...]) + bq_ref[...]
    o_ref[1] = _mmT(k_ref[...], wk_ref[...]) + bk_ref[...]
    ov_ref[...] = (_mmT(v_ref[...], wv_ref[...])
                   + bv_ref[...]).astype(jnp.bfloat16)


def _qkv_proj(x_q, x_k, x_v, wq, wk, wv, bq, bk, bv, bm,
              interpret=False):
    M, D = x_q.shape
    row = pl.BlockSpec((bm, D), lambda j: (j, 0))
    full = pl.BlockSpec((D, D), lambda j: (0, 0))
    vec = pl.BlockSpec((1, D), lambda j: (0, 0))
    return pl.pallas_call(
        _qkv_kernel,
        grid=(M // bm,),
        in_specs=[row, row, row, full, full, full, vec, vec, vec],
        out_specs=[
            pl.BlockSpec((2, bm, D), lambda j: (0, j, 0)),
            pl.BlockSpec((bm, D), lambda j: (j, 0)),
        ],
        out_shape=[
            jax.ShapeDtypeStruct((2, M, D), jnp.float32),
            jax.ShapeDtypeStruct((M, D), jnp.bfloat16),
        ],
        compiler_params=pltpu.CompilerParams(
            dimension_semantics=("parallel",)),
        interpret=interpret,
    )(x_q, x_k, x_v, wq, wk, wv,
      bq.reshape(1, D), bk.reshape(1, D), bv.reshape(1, D))


def _rank_key(x):
    b = jax.lax.bitcast_convert_type(x, jnp.int32)
    k = jnp.where(b >= 0, b, b ^ jnp.int32(0x7FFFFFFF))
    ii = jax.lax.broadcasted_iota(jnp.int32, x.shape, 1)
    return (k & jnp.int32(-32)) | (jnp.int32(C - 1) - ii)


def _first_max_mask_ax1(x):
    key = _rank_key(x)
    m = jnp.max(key, axis=1, keepdims=True)
    return key == m


def _topk_softmax3(sc3):
    gmax = jnp.max(sc3, axis=1, keepdims=True)
    work = _rank_key(sc3)
    neg = jnp.int32(-2147483648)
    selm = jnp.zeros(sc3.shape, dtype=jnp.bool_)
    for _ in range(TOPK):
        m = jnp.max(work, axis=1, keepdims=True)
        f = work == m
        selm = jnp.logical_or(selm, f)
        work = jnp.where(f, neg, work)
    e = jnp.where(selm, jnp.exp(sc3 - gmax), 0.0)
    return e / jnp.sum(e, axis=1, keepdims=True)


def _block_diag(mats):
    z = jnp.zeros(mats[0].shape, dtype=mats[0].dtype)
    rows = []
    for i, a in enumerate(mats):
        rows.append(jnp.concatenate(
            [a if j == i else z for j in range(len(mats))], axis=1))
    return jnp.concatenate(rows, axis=0)


def _fused_kernel(scale, hd, q_ref, k_ref, v_ref, c_ref, wo_ref, bo_ref,
                  aw_ref, o_ref):
    h = pl.program_id(1)
    qb = q_ref[0]
    kb = k_ref[0]
    vb = v_ref[...]
    S = kb.shape[0]

    lane = jax.lax.broadcasted_iota(jnp.int32, kb.shape, 1)
    head_of_lane = lane // hd
    sq = kb * kb
    nrm = jnp.zeros(kb.shape, dtype=jnp.float32)
    for i in range(G):
        sel = head_of_lane == i
        ni = jnp.sum(jnp.where(sel, sq, 0.0), axis=1, keepdims=True)
        nrm = jnp.where(sel, ni, nrm)
    nrm = jnp.maximum(jnp.sqrt(nrm), 1e-12)
    kn = kb / nrm

    cens = [c_ref[i] for i in range(G)]
    cns = [c / jnp.maximum(
        jnp.sqrt(jnp.sum(c * c, axis=-1, keepdims=True)), 1e-12)
        for c in cens]
    cnD = _block_diag(cns)

    simsT = jax.lax.dot_general(
        cnD, kn, (((1,), (1,)), ((), ())),
        preferred_element_type=jnp.float32)
    sims3 = simsT.reshape(G, C, S)

    oh3 = _first_max_mask_ax1(sims3).astype(jnp.float32)
    counts = jnp.sum(oh3, axis=2, keepdims=True)
    ohT = oh3.reshape(G * C, S)
    ksumD = jax.lax.dot_general(
        ohT, kb, (((1,), (0,)), ((), ())),
        preferred_element_type=jnp.float32)
    vsumD = jax.lax.dot_general(
        ohT.astype(jnp.bfloat16), vb, (((1,), (0,)), ((), ())),
        preferred_element_type=jnp.float32)

    has = counts > 0.0
    den = jnp.maximum(counts, 1.0)
    kmeans = []
    vmeans = []
    for i in range(G):
        ks = ksumD[C * i:C * (i + 1), hd * i:hd * (i + 1)]
        vs = vsumD[C * i:C * (i + 1), hd * i:hd * (i + 1)]
        kmeans.append(jnp.where(has[i], ks / den[i], cens[i]))
        vmeans.append(jnp.where(has[i], vs / den[i], 0.0))

    kmD = _block_diag(kmeans)
    scoresT = jax.lax.dot_general(
        kmD, qb, (((1,), (1,)), ((), ())),
        preferred_element_type=jnp.float32) * scale

    aw3 = _topk_softmax3(scoresT.reshape(G, C, S))

    awT = aw3.reshape(G * C, S)
    vmD = _block_diag(vmeans)
    y = jax.lax.dot_general(
        awT, vmD, (((0,), (0,)), ((), ())),
        preferred_element_type=jnp.float32)
    partial = jax.lax.dot_general(
        y.astype(jnp.bfloat16), wo_ref[...].astype(jnp.bfloat16),
        (((1,), (1,)), ((), ())),
        preferred_element_type=jnp.float32)

    for i in range(G):
        aw_ref[0, i] = aw3[i].T

    @pl.when(h == 0)
    def _():
        o_ref[0] = partial + bo_ref[...]

    @pl.when(h != 0)
    def _():
        o_ref[0] += partial


def _fused_attn(qk, vproj, cen, wo, bo, B, S, scale, interpret=False):
    C_, hd = cen.shape[1], cen.shape[2]
    D = qk.shape[2]
    return pl.pallas_call(
        functools.partial(_fused_kernel, scale, hd),
        grid=(B, H // G),
        in_specs=[
            pl.BlockSpec((1, S, G * hd), lambda b, h: (0, b, h)),
            pl.BlockSpec((1, S, G * hd), lambda b, h: (1, b, h)),
            pl.BlockSpec((S, G * hd), lambda b, h: (b, h)),
            pl.BlockSpec((G, C_, hd), lambda b, h: (h, 0, 0)),
            pl.BlockSpec((D, G * hd), lambda b, h: (0, h)),
            pl.BlockSpec((1, D), lambda b, h: (0, 0)),
        ],
        out_specs=[
            pl.BlockSpec((1, G, S, C_), lambda b, h: (b, h, 0, 0)),
            pl.BlockSpec((1, S, D), lambda b, h: (b, 0, 0)),
        ],
        out_shape=[
            jax.ShapeDtypeStruct((B, H, S, C_), jnp.float32),
            jax.ShapeDtypeStruct((B, S, D), jnp.float32),
        ],
        compiler_params=pltpu.CompilerParams(
            dimension_semantics=("parallel", "arbitrary")),
        interpret=interpret,
    )(qk, qk, vproj, cen, wo, bo.reshape(1, D))


def _impl(query, key, value, Wq, bq, Wk, bk, Wv, bv, Wo, bo, centroids,
          interpret=False):
    B, Qlen, D = query.shape
    S = key.shape[1]
    hd = D // H
    scale = hd ** (-0.5)

    qk, vproj = _qkv_proj(query.reshape(B * Qlen, D),
                          key.reshape(B * S, D),
                          value.reshape(B * S, D), Wq, Wk, Wv,
                          bq, bk, bv, bm=1024, interpret=interpret)
    attn, out = _fused_attn(qk, vproj, centroids,
                            Wo, bo, B, S, scale,
                            interpret=interpret)
    return out, attn


def kernel(query, key, value, Wq, bq, Wk, bk, Wv, bv, Wo, bo, centroids):
    return _impl(query, key, value, Wq, bq, Wk, bk, Wv, bv, Wo, bo,
                 centroids)


# speedup vs baseline: 1.2619x; 1.0019x over previous
import functools

import jax
import jax.numpy as jnp
from jax.experimental import pallas as pl
from jax.experimental.pallas import tpu as pltpu

H = 16
C = 32
TOPK = 8
G = 4


def _mmT(x, w):
    return jax.lax.dot_general(
        x, w, (((1,), (1,)), ((), ())),
        preferred_element_type=jnp.float32)


def _qkv_kernel(q_ref, k_ref, v_ref, wq_ref, wk_ref, wv_ref,
                bq_ref, bk_ref, bv_ref, o_ref, ov_ref):
    o_ref[0] = _mmT(q_ref[...], wq_ref[...]) + bq_ref[...]
    o_ref[1] = _mmT(k_ref[...], wk_ref[...]) + bk_ref[...]
    ov_ref[...] = (_mmT(v_ref[...], wv_ref[...])
                   + bv_ref[...]).astype(jnp.bfloat16)


def _qkv_proj(x_q, x_k, x_v, wq, wk, wv, bq, bk, bv, bm,
              interpret=False):
    M, D = x_q.shape
    row = pl.BlockSpec((bm, D), lambda j: (j, 0))
    full = pl.BlockSpec((D, D), lambda j: (0, 0))
    vec = pl.BlockSpec((1, D), lambda j: (0, 0))
    return pl.pallas_call(
        _qkv_kernel,
        grid=(M // bm,),
        in_specs=[row, row, row, full, full, full, vec, vec, vec],
        out_specs=[
            pl.BlockSpec((2, bm, D), lambda j: (0, j, 0)),
            pl.BlockSpec((bm, D), lambda j: (j, 0)),
        ],
        out_shape=[
            jax.ShapeDtypeStruct((2, M, D), jnp.float32),
            jax.ShapeDtypeStruct((M, D), jnp.bfloat16),
        ],
        compiler_params=pltpu.CompilerParams(
            dimension_semantics=("parallel",)),
        interpret=interpret,
    )(x_q, x_k, x_v, wq, wk, wv,
      bq.reshape(1, D), bk.reshape(1, D), bv.reshape(1, D))


def _rank_key(x):
    b = jax.lax.bitcast_convert_type(x, jnp.int32)
    k = jnp.where(b >= 0, b, b ^ jnp.int32(0x7FFFFFFF))
    ii = jax.lax.broadcasted_iota(jnp.int32, x.shape, 1)
    return (k & jnp.int32(-32)) | (jnp.int32(C - 1) - ii)


def _first_max_mask_ax1(x):
    key = _rank_key(x)
    m = jnp.max(key, axis=1, keepdims=True)
    return key == m


def _topk_softmax3(sc3):
    gmax = jnp.max(sc3, axis=1, keepdims=True)
    work = _rank_key(sc3)
    neg = jnp.int32(-2147483648)
    selm = jnp.zeros(sc3.shape, dtype=jnp.bool_)
    for _ in range(TOPK):
        m = jnp.max(work, axis=1, keepdims=True)
        f = work == m
        selm = jnp.logical_or(selm, f)
        work = jnp.where(f, neg, work)
    e = jnp.where(selm, jnp.exp(sc3 - gmax), 0.0)
    return e * (1.0 / jnp.sum(e, axis=1, keepdims=True))


def _block_diag(mats):
    z = jnp.zeros(mats[0].shape, dtype=mats[0].dtype)
    rows = []
    for i, a in enumerate(mats):
        rows.append(jnp.concatenate(
            [a if j == i else z for j in range(len(mats))], axis=1))
    return jnp.concatenate(rows, axis=0)


def _fused_kernel(scale, hd, q_ref, k_ref, v_ref, c_ref, wo_ref, bo_ref,
                  aw_ref, o_ref):
    h = pl.program_id(1)
    qb = q_ref[0]
    kb = k_ref[0]
    vb = v_ref[...]
    S = kb.shape[0]

    lane = jax.lax.broadcasted_iota(jnp.int32, kb.shape, 1)
    head_of_lane = lane // hd
    sq = kb * kb
    nrm = jnp.zeros(kb.shape, dtype=jnp.float32)
    for i in range(G):
        sel = head_of_lane == i
        ni = jnp.sum(jnp.where(sel, sq, 0.0), axis=1, keepdims=True)
        nrm = jnp.where(sel, ni, nrm)
    nrm = jnp.maximum(jnp.sqrt(nrm), 1e-12)
    kn = kb / nrm

    cens = [c_ref[i] for i in range(G)]
    cns = [c / jnp.maximum(
        jnp.sqrt(jnp.sum(c * c, axis=-1, keepdims=True)), 1e-12)
        for c in cens]
    cnD = _block_diag(cns)

    simsT = jax.lax.dot_general(
        cnD, kn, (((1,), (1,)), ((), ())),
        preferred_element_type=jnp.float32)
    sims3 = simsT.reshape(G, C, S)

    oh3 = _first_max_mask_ax1(sims3).astype(jnp.float32)
    counts = jnp.sum(oh3, axis=2, keepdims=True)
    ohT = oh3.reshape(G * C, S)
    ksumD = jax.lax.dot_general(
        ohT, kb, (((1,), (0,)), ((), ())),
        preferred_element_type=jnp.float32)
    vsumD = jax.lax.dot_general(
        ohT.astype(jnp.bfloat16), vb, (((1,), (0,)), ((), ())),
        preferred_element_type=jnp.float32)

    has = counts > 0.0
    den = jnp.maximum(counts, 1.0)
    kmeans = []
    vmeans = []
    for i in range(G):
        ks = ksumD[C * i:C * (i + 1), hd * i:hd * (i + 1)]
        vs = vsumD[C * i:C * (i + 1), hd * i:hd * (i + 1)]
        kmeans.append(jnp.where(has[i], ks / den[i], cens[i]))
        vmeans.append(jnp.where(has[i], vs / den[i], 0.0))

    kmD = _block_diag(kmeans)
    scoresT = jax.lax.dot_general(
        kmD, qb, (((1,), (1,)), ((), ())),
        preferred_element_type=jnp.float32) * scale

    aw3 = _topk_softmax3(scoresT.reshape(G, C, S))

    awT = aw3.reshape(G * C, S)
    vmD = _block_diag(vmeans)
    y = jax.lax.dot_general(
        awT, vmD, (((0,), (0,)), ((), ())),
        preferred_element_type=jnp.float32)
    partial = jax.lax.dot_general(
        y.astype(jnp.bfloat16), wo_ref[...].astype(jnp.bfloat16),
        (((1,), (1,)), ((), ())),
        preferred_element_type=jnp.float32)

    for i in range(G):
        aw_ref[0, i] = aw3[i].T

    @pl.when(h == 0)
    def _():
        o_ref[0] = partial + bo_ref[...]

    @pl.when(h != 0)
    def _():
        o_ref[0] += partial


def _fused_attn(qk, vproj, cen, wo, bo, B, S, scale, interpret=False):
    C_, hd = cen.shape[1], cen.shape[2]
    D = qk.shape[2]
    return pl.pallas_call(
        functools.partial(_fused_kernel, scale, hd),
        grid=(B, H // G),
        in_specs=[
            pl.BlockSpec((1, S, G * hd), lambda b, h: (0, b, h)),
            pl.BlockSpec((1, S, G * hd), lambda b, h: (1, b, h)),
            pl.BlockSpec((S, G * hd), lambda b, h: (b, h)),
            pl.BlockSpec((G, C_, hd), lambda b, h: (h, 0, 0)),
            pl.BlockSpec((D, G * hd), lambda b, h: (0, h)),
            pl.BlockSpec((1, D), lambda b, h: (0, 0)),
        ],
        out_specs=[
            pl.BlockSpec((1, G, S, C_), lambda b, h: (b, h, 0, 0)),
            pl.BlockSpec((1, S, D), lambda b, h: (b, 0, 0)),
        ],
        out_shape=[
            jax.ShapeDtypeStruct((B, H, S, C_), jnp.float32),
            jax.ShapeDtypeStruct((B, S, D), jnp.float32),
        ],
        compiler_params=pltpu.CompilerParams(
            dimension_semantics=("parallel", "arbitrary")),
        interpret=interpret,
    )(qk, qk, vproj, cen, wo, bo.reshape(1, D))


def _impl(query, key, value, Wq, bq, Wk, bk, Wv, bv, Wo, bo, centroids,
          interpret=False):
    B, Qlen, D = query.shape
    S = key.shape[1]
    hd = D // H
    scale = hd ** (-0.5)

    qk, vproj = _qkv_proj(query.reshape(B * Qlen, D),
                          key.reshape(B * S, D),
                          value.reshape(B * S, D), Wq, Wk, Wv,
                          bq, bk, bv, bm=1024, interpret=interpret)
    attn, out = _fused_attn(qk, vproj, centroids,
                            Wo, bo, B, S, scale,
                            interpret=interpret)
    return out, attn


def kernel(query, key, value, Wq, bq, Wk, bk, Wv, bv, Wo, bo, centroids):
    return _impl(query, key, value, Wq, bq, Wk, bk, Wv, bv, Wo, bo,
                 centroids)
